# baseline (speedup 1.0000x reference)
import functools

import jax
import jax.numpy as jnp
from jax import lax
from jax.experimental import pallas as pl
from jax.experimental.pallas import tpu as pltpu
from jax.experimental.pallas import tpu_sc as plsc

VOCAB = 32320
DIM = 1024
B = 4
S = 2048

_INFO = plsc.get_sparse_core_info()
_NC, _NS = _INFO.num_cores, _INFO.num_subcores
_NW = _NC * _NS
_N_IDX = B * S
_PER_W = _N_IDX // _NW
_W_PER_ROW = S // _PER_W
_CHUNK = 16
_NCHUNK = _PER_W // _CHUNK
_NBUF = 4


@functools.partial(
    pl.kernel,
    out_type=(
        jax.ShapeDtypeStruct((_N_IDX, DIM), jnp.float32),
        jax.ShapeDtypeStruct((B, S), jnp.float32),
        jax.ShapeDtypeStruct((B, S), jnp.float32),
    ),
    mesh=plsc.VectorSubcoreMesh(core_axis_name="c", subcore_axis_name="s"),
    scratch_types=(
        [pltpu.VMEM((_PER_W,), jnp.int32)]
        + [pltpu.VMEM((_CHUNK, DIM), jnp.float32)] * _NBUF
        + [pltpu.SemaphoreType.DMA] * (2 * _NBUF + 1)
    ),
)
def _gather_rows(idx_hbm, in1_hbm, in2_hbm, table_hbm,
                 out_hbm, o1_hbm, o2_hbm, idx_v, *bufs_and_sems):
    bufs = bufs_and_sems[:_NBUF]
    gsems = bufs_and_sems[_NBUF:2 * _NBUF]
    ssems = bufs_and_sems[2 * _NBUF:3 * _NBUF]
    psem = bufs_and_sems[3 * _NBUF]
    wid = lax.axis_index("s") * _NC + lax.axis_index("c")
    base = wid * _PER_W
    row = wid // _W_PER_ROW
    col = (wid % _W_PER_ROW) * _PER_W

    @pl.when(wid == 0)
    def _():
        pltpu.make_async_copy(in1_hbm, o1_hbm, psem).start()

    @pl.when(wid == 1)
    def _():
        pltpu.make_async_copy(in2_hbm, o2_hbm, psem).start()

    pltpu.sync_copy(idx_hbm.at[row, pl.ds(col, _PER_W)], idx_v)

    def gather(c):
        return pltpu.async_copy(
            table_hbm.at[idx_v.at[pl.ds(c * _CHUNK, _CHUNK)]],
            bufs[c % _NBUF], gsems[c % _NBUF])

    def store(c):
        return pltpu.async_copy(
            bufs[c % _NBUF], out_hbm.at[pl.ds(base + c * _CHUNK, _CHUNK)],
            ssems[c % _NBUF])

    gathers = [None] * _NBUF
    stores = [None] * _NBUF
    for i in range(_NBUF - 1):
        gathers[i] = gather(i)
    for c in range(_NCHUNK):
        b = c % _NBUF
        nc = c + _NBUF - 1
        if nc < _NCHUNK:
            if c >= 1:
                stores[nc % _NBUF].wait()
            gathers[nc % _NBUF] = gather(nc)
        gathers[b].wait()
        stores[b] = store(c)
    for c in range(_NCHUNK - _NBUF, _NCHUNK):
        stores[c % _NBUF].wait()

    @pl.when(wid == 0)
    def _():
        pltpu.make_async_copy(in1_hbm, o1_hbm, psem).wait()

    @pl.when(wid == 1)
    def _():
        pltpu.make_async_copy(in2_hbm, o2_hbm, psem).wait()


def kernel(input0, input1, input2, W):
    idx = input0.astype(jnp.int32)
    rows, o1, o2 = _gather_rows(idx, input1, input2, W)
    return (o1, o2, rows.reshape(B, S, DIM))


# speedup vs baseline: 1.0255x; 1.0255x over previous
import functools

import jax
import jax.numpy as jnp
from jax import lax
from jax.experimental import pallas as pl
from jax.experimental.pallas import tpu as pltpu
from jax.experimental.pallas import tpu_sc as plsc

VOCAB = 32320
DIM = 1024
B = 4
S = 2048

_INFO = plsc.get_sparse_core_info()
_NC, _NS = _INFO.num_cores, _INFO.num_subcores
_NW = _NC * _NS
_N_IDX = B * S
_PER_W = _N_IDX // _NW
_W_PER_ROW = S // _PER_W
_CHUNK = 16
_NCHUNK = _PER_W // _CHUNK
_NBUF = 6


@functools.partial(
    pl.kernel,
    out_type=(
        jax.ShapeDtypeStruct((_N_IDX, DIM), jnp.float32),
        jax.ShapeDtypeStruct((B, S), jnp.float32),
        jax.ShapeDtypeStruct((B, S), jnp.float32),
    ),
    mesh=plsc.VectorSubcoreMesh(core_axis_name="c", subcore_axis_name="s"),
    scratch_types=(
        [pltpu.VMEM((_PER_W,), jnp.int32)]
        + [pltpu.VMEM((_CHUNK, DIM), jnp.float32)] * _NBUF
        + [pltpu.SemaphoreType.DMA] * (2 * _NBUF + 1)
    ),
)
def _gather_rows(idx_hbm, in1_hbm, in2_hbm, table_hbm,
                 out_hbm, o1_hbm, o2_hbm, idx_v, *bufs_and_sems):
    bufs = bufs_and_sems[:_NBUF]
    gsems = bufs_and_sems[_NBUF:2 * _NBUF]
    ssems = bufs_and_sems[2 * _NBUF:3 * _NBUF]
    psem = bufs_and_sems[3 * _NBUF]
    wid = lax.axis_index("s") * _NC + lax.axis_index("c")
    base = wid * _PER_W
    row = wid // _W_PER_ROW
    col = (wid % _W_PER_ROW) * _PER_W

    @pl.when(wid == 0)
    def _():
        pltpu.make_async_copy(in1_hbm, o1_hbm, psem).start()

    @pl.when(wid == 1)
    def _():
        pltpu.make_async_copy(in2_hbm, o2_hbm, psem).start()

    pltpu.sync_copy(idx_hbm.at[row, pl.ds(col, _PER_W)], idx_v)

    def gather(c):
        return pltpu.async_copy(
            table_hbm.at[idx_v.at[pl.ds(c * _CHUNK, _CHUNK)]],
            bufs[c % _NBUF], gsems[c % _NBUF])

    def store(c):
        return pltpu.async_copy(
            bufs[c % _NBUF], out_hbm.at[pl.ds(base + c * _CHUNK, _CHUNK)],
            ssems[c % _NBUF])

    gathers = [None] * _NBUF
    stores = [None] * _NBUF
    for i in range(_NBUF - 1):
        gathers[i] = gather(i)
    for c in range(_NCHUNK):
        b = c % _NBUF
        nc = c + _NBUF - 1
        if nc < _NCHUNK:
            if c >= 1:
                stores[nc % _NBUF].wait()
            gathers[nc % _NBUF] = gather(nc)
        gathers[b].wait()
        stores[b] = store(c)
    for c in range(_NCHUNK - _NBUF, _NCHUNK):
        stores[c % _NBUF].wait()

    @pl.when(wid == 0)
    def _():
        pltpu.make_async_copy(in1_hbm, o1_hbm, psem).wait()

    @pl.when(wid == 1)
    def _():
        pltpu.make_async_copy(in2_hbm, o2_hbm, psem).wait()


def kernel(input0, input1, input2, W):
    idx = input0.astype(jnp.int32)
    rows, o1, o2 = _gather_rows(idx, input1, input2, W)
    return (o1, o2, rows.reshape(B, S, DIM))
